# Initial kernel scaffold; baseline (speedup 1.0000x reference)
import jax
import jax.numpy as jnp
from jax.experimental import pallas as pl


def kernel(xyz, x, knn, W_proj, W1, b1, W2, b2, W3, b3, W4, b4, gamma, beta):
    raise NotImplementedError("write your pallas kernel here")


# speedup vs baseline: 24.1580x; 24.1580x over previous
import functools

import jax
import jax.numpy as jnp
from jax import lax
from jax.experimental import pallas as pl
from jax.experimental.pallas import tpu as pltpu
from jax.experimental.pallas import tpu_sc as plsc

B, N, K = 2, 10000, 16
IN_DIM, OUT_DIM = 128, 128
H = OUT_DIM // 2
ROWS = B * N
TOTAL = ROWS * K
HALF = OUT_DIM // 2


_PROJ_R = 2000


def _proj_body(x_ref, w_ref, xyz_ref, xpf_ref, tbl_ref):
    xp = jnp.dot(x_ref[...], w_ref[...], preferred_element_type=jnp.float32)
    xpf_ref[...] = xp
    be = lax.bitcast_convert_type(xp[:, :HALF], jnp.int32) + 0x8000
    bo = lax.bitcast_convert_type(xp[:, HALF:], jnp.int32) + 0x8000
    word = (bo & jnp.int32(-65536)) | lax.shift_right_logical(be, 16)
    zword = lax.bitcast_convert_type(xyz_ref[...], jnp.int32)
    tbl_ref[...] = jnp.concatenate([word, zword], axis=1)


def _proj(x2d, w_perm, xyz64):
    return pl.pallas_call(
        _proj_body,
        grid=(ROWS // _PROJ_R,),
        in_specs=[
            pl.BlockSpec((_PROJ_R, IN_DIM), lambda i: (i, 0)),
            pl.BlockSpec((IN_DIM, OUT_DIM), lambda i: (0, 0)),
            pl.BlockSpec((_PROJ_R, HALF), lambda i: (i, 0)),
        ],
        out_specs=[
            pl.BlockSpec((_PROJ_R, OUT_DIM), lambda i: (i, 0)),
            pl.BlockSpec((_PROJ_R, OUT_DIM), lambda i: (i, 0)),
        ],
        out_shape=[
            jax.ShapeDtypeStruct((ROWS, OUT_DIM), jnp.float32),
            jax.ShapeDtypeStruct((ROWS, OUT_DIM), jnp.int32),
        ],
    )(x2d, w_perm, xyz64)


_NC = 2
_NS = 16
_NW = _NC * _NS
_PER_W = TOTAL // _NW
_CH = 400
_NCHUNK = _PER_W // _CH


def _sc_gather_body(tbl_hbm, idx_hbm, out_hbm, idx_v, rows_v, sem):
    wid = lax.axis_index("s") * _NC + lax.axis_index("c")

    def chunk(c, carry):
        base = wid * _PER_W + c * _CH
        pltpu.sync_copy(idx_hbm.at[pl.ds(base, _CH)], idx_v)
        pltpu.async_copy(tbl_hbm.at[idx_v], rows_v, sem).wait()
        pltpu.sync_copy(rows_v, out_hbm.at[pl.ds(base, _CH)])
        return carry

    lax.fori_loop(0, _NCHUNK, chunk, 0)


def _sc_gather(tbl, idx_flat):
    mesh = plsc.VectorSubcoreMesh(core_axis_name="c", subcore_axis_name="s")
    fn = functools.partial(
        pl.kernel,
        mesh=mesh,
        out_type=jax.ShapeDtypeStruct((TOTAL, OUT_DIM), jnp.int32),
        scratch_types=[
            pltpu.VMEM((_CH,), jnp.int32),
            pltpu.VMEM((_CH, OUT_DIM), jnp.int32),
            pltpu.SemaphoreType.DMA,
        ],
    )(_sc_gather_body)
    return fn(tbl, idx_flat)


_P = 400
_PK = _P * K


def _main_body(gt_ref, xpf_ref, cz_ref,
               w1_ref, b1_ref, w2_ref, b2_ref, w3a_ref, w3b_ref, b3_ref,
               w4_ref, b4_ref,
               out_ref, sum_ref, sq_ref):
    f32 = jnp.float32
    w = gt_ref[...]
    wx = w[:, :HALF]
    lo = lax.bitcast_convert_type(wx << 16, f32)
    hi = lax.bitcast_convert_type(wx & jnp.int32(-65536), f32)
    gx = jnp.concatenate([lo, hi], axis=1)
    zg = lax.bitcast_convert_type(w[:, HALF:], f32)

    cz = cz_ref[...]
    z = zg.reshape(_P, K, HALF) - cz[:, None, :]
    zf = z.reshape(_PK, HALF)
    f0 = jnp.dot(zf, w1_ref[...], preferred_element_type=f32) + b1_ref[...]
    p_local = jnp.max(f0.reshape(_P, K, H), axis=1)
    f1 = jnp.dot(f0, w2_ref[...], preferred_element_type=f32) + b2_ref[...]
    q = jnp.dot(p_local, w3b_ref[...], preferred_element_type=f32) + b3_ref[...]
    pre = (jnp.dot(f1, w3a_ref[...], preferred_element_type=f32)
           .reshape(_P, K, OUT_DIM) + q[:, None, :])
    act = (0.5 * pre * (1.0 + lax.erf(pre * jnp.float32(0.7071067811865476)))
           ).reshape(_PK, OUT_DIM)
    s = (jnp.dot(act, w4_ref[...], preferred_element_type=f32)
         + b4_ref[...] + gx)
    m = jnp.max(s.reshape(_P, K, OUT_DIM), axis=1) - xpf_ref[...]
    out_ref[...] = m

    @pl.when(pl.program_id(0) == 0)
    def _init():
        sum_ref[...] = jnp.zeros_like(sum_ref)
        sq_ref[...] = jnp.zeros_like(sq_ref)

    sum_ref[...] += jnp.sum(m, axis=0, keepdims=True)
    sq_ref[...] += jnp.sum(m * m, axis=0, keepdims=True)


def _main(gt, xpf, xyz64, w1p, b1, w2, b2, w3a, w3b, b3, w4p, b4p):
    grid = (ROWS // _P,)
    return pl.pallas_call(
        _main_body,
        grid=grid,
        in_specs=[
            pl.BlockSpec((_PK, OUT_DIM), lambda i: (i, 0)),
            pl.BlockSpec((_P, OUT_DIM), lambda i: (i, 0)),
            pl.BlockSpec((_P, HALF), lambda i: (i, 0)),
            pl.BlockSpec((HALF, H), lambda i: (0, 0)),
            pl.BlockSpec((1, H), lambda i: (0, 0)),
            pl.BlockSpec((H, H), lambda i: (0, 0)),
            pl.BlockSpec((1, H), lambda i: (0, 0)),
            pl.BlockSpec((H, OUT_DIM), lambda i: (0, 0)),
            pl.BlockSpec((H, OUT_DIM), lambda i: (0, 0)),
            pl.BlockSpec((1, OUT_DIM), lambda i: (0, 0)),
            pl.BlockSpec((OUT_DIM, OUT_DIM), lambda i: (0, 0)),
            pl.BlockSpec((1, OUT_DIM), lambda i: (0, 0)),
        ],
        out_specs=[
            pl.BlockSpec((_P, OUT_DIM), lambda i: (i, 0)),
            pl.BlockSpec((1, OUT_DIM), lambda i: (0, 0)),
            pl.BlockSpec((1, OUT_DIM), lambda i: (0, 0)),
        ],
        out_shape=[
            jax.ShapeDtypeStruct((ROWS, OUT_DIM), jnp.float32),
            jax.ShapeDtypeStruct((1, OUT_DIM), jnp.float32),
            jax.ShapeDtypeStruct((1, OUT_DIM), jnp.float32),
        ],
    )(gt, xpf, xyz64, w1p, b1, w2, b2, w3a, w3b, b3, w4p, b4p)


_BN_R = 2000


def _bn_body(m_ref, sum_ref, sq_ref, g_ref, b_ref, out_ref):
    mean = sum_ref[...] * (1.0 / ROWS)
    var = sq_ref[...] * (1.0 / ROWS) - mean * mean
    inv = lax.rsqrt(var + 1e-5) * g_ref[...]
    out_ref[...] = (m_ref[...] - mean) * inv + b_ref[...]


def _bn(m, s, sq, gamma, beta):
    return pl.pallas_call(
        _bn_body,
        grid=(ROWS // _BN_R,),
        in_specs=[
            pl.BlockSpec((_BN_R, OUT_DIM), lambda i: (i, 0)),
            pl.BlockSpec((1, OUT_DIM), lambda i: (0, 0)),
            pl.BlockSpec((1, OUT_DIM), lambda i: (0, 0)),
            pl.BlockSpec((1, OUT_DIM), lambda i: (0, 0)),
            pl.BlockSpec((1, OUT_DIM), lambda i: (0, 0)),
        ],
        out_specs=pl.BlockSpec((_BN_R, OUT_DIM), lambda i: (i, 0)),
        out_shape=jax.ShapeDtypeStruct((ROWS, OUT_DIM), jnp.float32),
    )(m, s, sq, gamma, beta)


def kernel(xyz, x, knn, W_proj, W1, b1, W2, b2, W3, b3, W4, b4, gamma, beta):
    perm = jnp.concatenate([jnp.arange(0, OUT_DIM, 2),
                            jnp.arange(1, OUT_DIM, 2)])
    x2d = x.reshape(ROWS, IN_DIM)
    xyz64 = jnp.pad(xyz.reshape(ROWS, 3), ((0, 0), (0, HALF - 3)))
    idx_flat = (knn + (jnp.arange(B, dtype=jnp.int32) * N)[:, None, None]
                ).reshape(TOTAL)

    xpf, tbl = _proj(x2d, W_proj[:, perm], xyz64)
    gt = _sc_gather(tbl, idx_flat)

    w1p = jnp.pad(W1, ((0, HALF - 3), (0, 0)))
    w3a, w3b = W3[:H, :], W3[H:, :]
    m, s, sq = _main(gt, xpf, xyz64,
                     w1p, b1.reshape(1, H), W2, b2.reshape(1, H),
                     w3a, w3b, b3.reshape(1, OUT_DIM),
                     W4[:, perm], b4[perm].reshape(1, OUT_DIM))
    outp = _bn(m, s, sq, gamma[perm].reshape(1, OUT_DIM),
               beta[perm].reshape(1, OUT_DIM))
    out = jnp.stack([outp[:, :HALF], outp[:, HALF:]], axis=-1)
    return out.reshape(B, N, OUT_DIM)


# speedup vs baseline: 24.2050x; 1.0019x over previous
import functools

import jax
import jax.numpy as jnp
from jax import lax
from jax.experimental import pallas as pl
from jax.experimental.pallas import tpu as pltpu
from jax.experimental.pallas import tpu_sc as plsc

B, N, K = 2, 10000, 16
IN_DIM, OUT_DIM = 128, 128
H = OUT_DIM // 2
ROWS = B * N
TOTAL = ROWS * K
HALF = OUT_DIM // 2


_PROJ_R = 2000


def _proj_body(x_ref, w_ref, xyz_ref, xpf_ref, tbl_ref):
    xp = jnp.dot(x_ref[...], w_ref[...], preferred_element_type=jnp.float32)
    xpf_ref[...] = xp
    be = lax.bitcast_convert_type(xp[:, :HALF], jnp.int32) + 0x8000
    bo = lax.bitcast_convert_type(xp[:, HALF:], jnp.int32) + 0x8000
    word = (bo & jnp.int32(-65536)) | lax.shift_right_logical(be, 16)
    zword = lax.bitcast_convert_type(xyz_ref[...], jnp.int32)
    tbl_ref[...] = jnp.concatenate([word, zword], axis=1)


def _proj(x2d, w_perm, xyz64):
    return pl.pallas_call(
        _proj_body,
        grid=(ROWS // _PROJ_R,),
        in_specs=[
            pl.BlockSpec((_PROJ_R, IN_DIM), lambda i: (i, 0)),
            pl.BlockSpec((IN_DIM, OUT_DIM), lambda i: (0, 0)),
            pl.BlockSpec((_PROJ_R, HALF), lambda i: (i, 0)),
        ],
        out_specs=[
            pl.BlockSpec((_PROJ_R, OUT_DIM), lambda i: (i, 0)),
            pl.BlockSpec((_PROJ_R, OUT_DIM), lambda i: (i, 0)),
        ],
        out_shape=[
            jax.ShapeDtypeStruct((ROWS, OUT_DIM), jnp.float32),
            jax.ShapeDtypeStruct((ROWS, OUT_DIM), jnp.int32),
        ],
    )(x2d, w_perm, xyz64)


_NC = 2
_NS = 16
_NW = _NC * _NS
_PER_W = TOTAL // _NW
_CH = 400
_NCHUNK = _PER_W // _CH


def _sc_gather_body(tbl_hbm, idx_hbm, out_hbm, idx_v, rows_v, sem):
    wid = lax.axis_index("s") * _NC + lax.axis_index("c")

    def chunk(c, carry):
        base = wid * _PER_W + c * _CH
        pltpu.sync_copy(idx_hbm.at[pl.ds(base, _CH)], idx_v)
        pltpu.async_copy(tbl_hbm.at[idx_v], rows_v, sem).wait()
        pltpu.sync_copy(rows_v, out_hbm.at[pl.ds(base, _CH)])
        return carry

    lax.fori_loop(0, _NCHUNK, chunk, 0)


def _sc_gather(tbl, idx_flat):
    mesh = plsc.VectorSubcoreMesh(core_axis_name="c", subcore_axis_name="s")
    fn = functools.partial(
        pl.kernel,
        mesh=mesh,
        out_type=jax.ShapeDtypeStruct((TOTAL, OUT_DIM), jnp.int32),
        scratch_types=[
            pltpu.VMEM((_CH,), jnp.int32),
            pltpu.VMEM((_CH, OUT_DIM), jnp.int32),
            pltpu.SemaphoreType.DMA,
        ],
    )(_sc_gather_body)
    return fn(tbl, idx_flat)


_P = 400
_PK = _P * K


def _main_body(gt_ref, xpf_ref, cz_ref,
               w1_ref, b1_ref, w2_ref, b2_ref, w3a_ref, w3b_ref, b3_ref,
               w4_ref, b4_ref,
               out_ref, sum_ref, sq_ref):
    f32 = jnp.float32
    w = gt_ref[...]
    wx = w[:, :HALF]
    lo = lax.bitcast_convert_type(wx << 16, f32)
    hi = lax.bitcast_convert_type(wx & jnp.int32(-65536), f32)
    gx = jnp.concatenate([lo, hi], axis=1)
    zg = lax.bitcast_convert_type(w[:, HALF:], f32)

    bf16 = jnp.bfloat16
    cz = cz_ref[...]
    z = zg.reshape(_P, K, HALF) - cz[:, None, :]
    zf = z.reshape(_PK, HALF)
    f0 = jnp.dot(zf.astype(bf16), w1_ref[...].astype(bf16),
                 preferred_element_type=f32) + b1_ref[...]
    p_local = jnp.max(f0.reshape(_P, K, H), axis=1)
    f1 = jnp.dot(f0.astype(bf16), w2_ref[...].astype(bf16),
                 preferred_element_type=f32) + b2_ref[...]
    q = jnp.dot(p_local.astype(bf16), w3b_ref[...].astype(bf16),
                preferred_element_type=f32) + b3_ref[...]
    pre = (jnp.dot(f1.astype(bf16), w3a_ref[...].astype(bf16),
                   preferred_element_type=f32)
           .reshape(_P, K, OUT_DIM) + q[:, None, :])
    act = (0.5 * pre * (1.0 + lax.erf(pre * jnp.float32(0.7071067811865476)))
           ).reshape(_PK, OUT_DIM)
    s = (jnp.dot(act.astype(bf16), w4_ref[...].astype(bf16),
                 preferred_element_type=f32)
         + b4_ref[...] + gx)
    m = jnp.max(s.reshape(_P, K, OUT_DIM), axis=1) - xpf_ref[...]
    out_ref[...] = m

    @pl.when(pl.program_id(0) == 0)
    def _init():
        sum_ref[...] = jnp.zeros_like(sum_ref)
        sq_ref[...] = jnp.zeros_like(sq_ref)

    sum_ref[...] += jnp.sum(m, axis=0, keepdims=True)
    sq_ref[...] += jnp.sum(m * m, axis=0, keepdims=True)


def _main(gt, xpf, xyz64, w1p, b1, w2, b2, w3a, w3b, b3, w4p, b4p):
    grid = (ROWS // _P,)
    return pl.pallas_call(
        _main_body,
        grid=grid,
        in_specs=[
            pl.BlockSpec((_PK, OUT_DIM), lambda i: (i, 0)),
            pl.BlockSpec((_P, OUT_DIM), lambda i: (i, 0)),
            pl.BlockSpec((_P, HALF), lambda i: (i, 0)),
            pl.BlockSpec((HALF, H), lambda i: (0, 0)),
            pl.BlockSpec((1, H), lambda i: (0, 0)),
            pl.BlockSpec((H, H), lambda i: (0, 0)),
            pl.BlockSpec((1, H), lambda i: (0, 0)),
            pl.BlockSpec((H, OUT_DIM), lambda i: (0, 0)),
            pl.BlockSpec((H, OUT_DIM), lambda i: (0, 0)),
            pl.BlockSpec((1, OUT_DIM), lambda i: (0, 0)),
            pl.BlockSpec((OUT_DIM, OUT_DIM), lambda i: (0, 0)),
            pl.BlockSpec((1, OUT_DIM), lambda i: (0, 0)),
        ],
        out_specs=[
            pl.BlockSpec((_P, OUT_DIM), lambda i: (i, 0)),
            pl.BlockSpec((1, OUT_DIM), lambda i: (0, 0)),
            pl.BlockSpec((1, OUT_DIM), lambda i: (0, 0)),
        ],
        out_shape=[
            jax.ShapeDtypeStruct((ROWS, OUT_DIM), jnp.float32),
            jax.ShapeDtypeStruct((1, OUT_DIM), jnp.float32),
            jax.ShapeDtypeStruct((1, OUT_DIM), jnp.float32),
        ],
    )(gt, xpf, xyz64, w1p, b1, w2, b2, w3a, w3b, b3, w4p, b4p)


_BN_R = 2000


def _bn_body(m_ref, sum_ref, sq_ref, g_ref, b_ref, out_ref):
    mean = sum_ref[...] * (1.0 / ROWS)
    var = sq_ref[...] * (1.0 / ROWS) - mean * mean
    inv = lax.rsqrt(var + 1e-5) * g_ref[...]
    out_ref[...] = (m_ref[...] - mean) * inv + b_ref[...]


def _bn(m, s, sq, gamma, beta):
    return pl.pallas_call(
        _bn_body,
        grid=(ROWS // _BN_R,),
        in_specs=[
            pl.BlockSpec((_BN_R, OUT_DIM), lambda i: (i, 0)),
            pl.BlockSpec((1, OUT_DIM), lambda i: (0, 0)),
            pl.BlockSpec((1, OUT_DIM), lambda i: (0, 0)),
            pl.BlockSpec((1, OUT_DIM), lambda i: (0, 0)),
            pl.BlockSpec((1, OUT_DIM), lambda i: (0, 0)),
        ],
        out_specs=pl.BlockSpec((_BN_R, OUT_DIM), lambda i: (i, 0)),
        out_shape=jax.ShapeDtypeStruct((ROWS, OUT_DIM), jnp.float32),
    )(m, s, sq, gamma, beta)


def kernel(xyz, x, knn, W_proj, W1, b1, W2, b2, W3, b3, W4, b4, gamma, beta):
    perm = jnp.concatenate([jnp.arange(0, OUT_DIM, 2),
                            jnp.arange(1, OUT_DIM, 2)])
    x2d = x.reshape(ROWS, IN_DIM)
    xyz64 = jnp.pad(xyz.reshape(ROWS, 3), ((0, 0), (0, HALF - 3)))
    idx_flat = (knn + (jnp.arange(B, dtype=jnp.int32) * N)[:, None, None]
                ).reshape(TOTAL)

    xpf, tbl = _proj(x2d, W_proj[:, perm], xyz64)
    gt = _sc_gather(tbl, idx_flat)

    w1p = jnp.pad(W1, ((0, HALF - 3), (0, 0)))
    w3a, w3b = W3[:H, :], W3[H:, :]
    m, s, sq = _main(gt, xpf, xyz64,
                     w1p, b1.reshape(1, H), W2, b2.reshape(1, H),
                     w3a, w3b, b3.reshape(1, OUT_DIM),
                     W4[:, perm], b4[perm].reshape(1, OUT_DIM))
    outp = _bn(m, s, sq, gamma[perm].reshape(1, OUT_DIM),
               beta[perm].reshape(1, OUT_DIM))
    out = jnp.stack([outp[:, :HALF], outp[:, HALF:]], axis=-1)
    return out.reshape(B, N, OUT_DIM)
